# baseline (speedup 1.0000x reference)
import jax
import jax.numpy as jnp
from jax.experimental import pallas as pl
from jax.experimental.pallas import tpu as pltpu

_N_INPUT = 28 * 28
_N_RES = 1000
_T = 25
_BETA = 0.95
_TH = 1.0

_K_PAD = 896
_N_PAD = 1024
_B_BLK = 512
_B_HALF = _B_BLK // 2


def _lsm_body(x_ref, win_ref, wh_ref, wl_ref, o_ref):
    win = win_ref[...]
    wh = wh_ref[...]
    wl = wl_ref[...]
    icA = jnp.dot(x_ref[:_B_HALF], win, preferred_element_type=jnp.float32)
    icB = jnp.dot(x_ref[_B_HALF:], win, preferred_element_type=jnp.float32)

    memA = icA
    spkA = (memA - _TH > 0).astype(jnp.bfloat16)
    ssA = spkA
    memB = icB
    spkB = (memB - _TH > 0).astype(jnp.bfloat16)
    ssB = spkB

    def step(_, c):
        memA, spkA, ssA, memB, spkB, ssB = c
        recA = (jnp.dot(spkA, wh, preferred_element_type=jnp.float32)
                + jnp.dot(spkA, wl, preferred_element_type=jnp.float32))
        recB = (jnp.dot(spkB, wh, preferred_element_type=jnp.float32)
                + jnp.dot(spkB, wl, preferred_element_type=jnp.float32))
        memA = _BETA * memA + (icA + recA) - spkA.astype(jnp.float32) * _TH
        memB = _BETA * memB + (icB + recB) - spkB.astype(jnp.float32) * _TH
        spkA = (memA - _TH > 0).astype(jnp.bfloat16)
        spkB = (memB - _TH > 0).astype(jnp.bfloat16)
        return memA, spkA, ssA + spkA, memB, spkB, ssB + spkB

    _, _, ssA, _, _, ssB = jax.lax.fori_loop(
        1, _T, step, (memA, spkA, ssA, memB, spkB, ssB))
    o_ref[:_B_HALF] = ssA.astype(jnp.float32) * (1.0 / _T)
    o_ref[_B_HALF:] = ssB.astype(jnp.float32) * (1.0 / _T)


def kernel(x, W_in, W_rec):
    B = x.shape[0]
    x_p = jnp.pad(x, ((0, 0), (0, _K_PAD - _N_INPUT)))
    win_t = jnp.pad(W_in.T, ((0, _K_PAD - _N_INPUT), (0, _N_PAD - _N_RES)))
    wrec_t = jnp.pad(W_rec.T, ((0, _N_PAD - _N_RES), (0, _N_PAD - _N_RES)))
    wrec_hi = wrec_t.astype(jnp.bfloat16)
    wrec_lo = (wrec_t - wrec_hi.astype(jnp.float32)).astype(jnp.bfloat16)

    out = pl.pallas_call(
        _lsm_body,
        out_shape=jax.ShapeDtypeStruct((B, _N_PAD), jnp.float32),
        grid=(B // _B_BLK,),
        in_specs=[
            pl.BlockSpec((_B_BLK, _K_PAD), lambda b: (b, 0)),
            pl.BlockSpec((_K_PAD, _N_PAD), lambda b: (0, 0)),
            pl.BlockSpec((_N_PAD, _N_PAD), lambda b: (0, 0)),
            pl.BlockSpec((_N_PAD, _N_PAD), lambda b: (0, 0)),
        ],
        out_specs=pl.BlockSpec((_B_BLK, _N_PAD), lambda b: (b, 0)),
        compiler_params=pltpu.CompilerParams(
            dimension_semantics=("parallel",),
        ),
        name="lsm_fused",
    )(x_p, win_t, wrec_hi, wrec_lo)
    return out[:, :_N_RES]


# speedup vs baseline: 1.2493x; 1.2493x over previous
import jax
import jax.numpy as jnp
from jax.experimental import pallas as pl
from jax.experimental.pallas import tpu as pltpu

_N_INPUT = 28 * 28
_N_RES = 1000
_T = 25
_BETA = 0.95
_TH = 1.0

_K_PAD = 896
_N_PAD = 1024
_B_BLK = 512
_B_HALF = _B_BLK // 2


def _lsm_body(x_ref, win_ref, wrec_ref, o_ref):
    icA = jnp.dot(x_ref[:_B_HALF], win_ref[...],
                  preferred_element_type=jnp.float32)
    icB = jnp.dot(x_ref[_B_HALF:], win_ref[...],
                  preferred_element_type=jnp.float32)

    memA = icA
    spkA = (memA - _TH > 0).astype(jnp.float32)
    ssA = spkA
    memB = icB
    spkB = (memB - _TH > 0).astype(jnp.float32)
    ssB = spkB

    def step(_, c):
        memA, spkA, ssA, memB, spkB, ssB = c
        recA = jnp.dot(spkA, wrec_ref[...], preferred_element_type=jnp.float32)
        recB = jnp.dot(spkB, wrec_ref[...], preferred_element_type=jnp.float32)
        memA = _BETA * memA + (icA + recA) - spkA * _TH
        memB = _BETA * memB + (icB + recB) - spkB * _TH
        spkA = (memA - _TH > 0).astype(jnp.float32)
        spkB = (memB - _TH > 0).astype(jnp.float32)
        return memA, spkA, ssA + spkA, memB, spkB, ssB + spkB

    _, _, ssA, _, _, ssB = jax.lax.fori_loop(
        1, _T, step, (memA, spkA, ssA, memB, spkB, ssB))
    o_ref[:_B_HALF] = ssA * (1.0 / _T)
    o_ref[_B_HALF:] = ssB * (1.0 / _T)


def kernel(x, W_in, W_rec):
    B = x.shape[0]
    x_p = jnp.pad(x, ((0, 0), (0, _K_PAD - _N_INPUT)))
    win_t = jnp.pad(W_in.T, ((0, _K_PAD - _N_INPUT), (0, _N_PAD - _N_RES)))
    wrec_t = jnp.pad(W_rec.T, ((0, _N_PAD - _N_RES), (0, _N_PAD - _N_RES)))

    out = pl.pallas_call(
        _lsm_body,
        out_shape=jax.ShapeDtypeStruct((B, _N_PAD), jnp.float32),
        grid=(B // _B_BLK,),
        in_specs=[
            pl.BlockSpec((_B_BLK, _K_PAD), lambda b: (b, 0)),
            pl.BlockSpec((_K_PAD, _N_PAD), lambda b: (0, 0)),
            pl.BlockSpec((_N_PAD, _N_PAD), lambda b: (0, 0)),
        ],
        out_specs=pl.BlockSpec((_B_BLK, _N_PAD), lambda b: (b, 0)),
        compiler_params=pltpu.CompilerParams(
            dimension_semantics=("parallel",),
        ),
        name="lsm_fused",
    )(x_p, win_t, wrec_t)
    return out[:, :_N_RES]


# speedup vs baseline: 1.2996x; 1.0402x over previous
import jax
import jax.numpy as jnp
from jax.experimental import pallas as pl
from jax.experimental.pallas import tpu as pltpu

_N_INPUT = 28 * 28
_N_RES = 1000
_T = 25
_BETA = 0.95
_TH = 1.0

_K_PAD = 896
_N_PAD = 1024
_B_BLK = 1024
_B_HALF = _B_BLK // 2


def _lsm_body(x_ref, win_ref, wrec_ref, o_ref):
    icA = jnp.dot(x_ref[:_B_HALF], win_ref[...],
                  preferred_element_type=jnp.float32)
    icB = jnp.dot(x_ref[_B_HALF:], win_ref[...],
                  preferred_element_type=jnp.float32)

    memA = icA
    spkA = (memA - _TH > 0).astype(jnp.float32)
    ssA = spkA
    memB = icB
    spkB = (memB - _TH > 0).astype(jnp.float32)
    ssB = spkB

    def step(_, c):
        memA, spkA, ssA, memB, spkB, ssB = c
        recA = jnp.dot(spkA, wrec_ref[...], preferred_element_type=jnp.float32)
        recB = jnp.dot(spkB, wrec_ref[...], preferred_element_type=jnp.float32)
        memA = _BETA * memA + (icA + recA) - spkA * _TH
        memB = _BETA * memB + (icB + recB) - spkB * _TH
        spkA = (memA - _TH > 0).astype(jnp.float32)
        spkB = (memB - _TH > 0).astype(jnp.float32)
        return memA, spkA, ssA + spkA, memB, spkB, ssB + spkB

    _, _, ssA, _, _, ssB = jax.lax.fori_loop(
        1, _T, step, (memA, spkA, ssA, memB, spkB, ssB))
    o_ref[:_B_HALF] = ssA * (1.0 / _T)
    o_ref[_B_HALF:] = ssB * (1.0 / _T)


def kernel(x, W_in, W_rec):
    B = x.shape[0]
    x_p = jnp.pad(x, ((0, 0), (0, _K_PAD - _N_INPUT)))
    win_t = jnp.pad(W_in.T, ((0, _K_PAD - _N_INPUT), (0, _N_PAD - _N_RES)))
    wrec_t = jnp.pad(W_rec.T, ((0, _N_PAD - _N_RES), (0, _N_PAD - _N_RES)))

    out = pl.pallas_call(
        _lsm_body,
        out_shape=jax.ShapeDtypeStruct((B, _N_PAD), jnp.float32),
        grid=(B // _B_BLK,),
        in_specs=[
            pl.BlockSpec((_B_BLK, _K_PAD), lambda b: (b, 0)),
            pl.BlockSpec((_K_PAD, _N_PAD), lambda b: (0, 0)),
            pl.BlockSpec((_N_PAD, _N_PAD), lambda b: (0, 0)),
        ],
        out_specs=pl.BlockSpec((_B_BLK, _N_PAD), lambda b: (b, 0)),
        compiler_params=pltpu.CompilerParams(
            dimension_semantics=("parallel",),
        ),
        name="lsm_fused",
    )(x_p, win_t, wrec_t)
    return out[:, :_N_RES]


# speedup vs baseline: 1.5179x; 1.1680x over previous
import jax
import jax.numpy as jnp
from jax.experimental import pallas as pl
from jax.experimental.pallas import tpu as pltpu

_N_INPUT = 28 * 28
_N_RES = 1000
_T = 25
_BETA = 0.95
_TH = 1.0

_K_PAD = 896
_N_PAD = 1024
_B_BLK = 1024
_B_HALF = _B_BLK // 2


def _lsm_body(x_ref, win_ref, wrec_ref, o_ref):
    icA = jnp.dot(x_ref[:_B_HALF], win_ref[...],
                  preferred_element_type=jnp.float32)
    icB = jnp.dot(x_ref[_B_HALF:], win_ref[...],
                  preferred_element_type=jnp.float32)

    memA = icA
    ssA = (memA - _TH > 0).astype(jnp.bfloat16)
    memB = icB
    ssB = (memB - _TH > 0).astype(jnp.bfloat16)

    def step(_, c):
        memA, ssA, memB, ssB = c
        spkA = (memA - _TH > 0).astype(jnp.float32)
        spkB = (memB - _TH > 0).astype(jnp.float32)
        recA = jnp.dot(spkA, wrec_ref[...], preferred_element_type=jnp.float32)
        recB = jnp.dot(spkB, wrec_ref[...], preferred_element_type=jnp.float32)
        memA = _BETA * memA + (icA + recA) - spkA * _TH
        memB = _BETA * memB + (icB + recB) - spkB * _TH
        ssA = ssA + (memA - _TH > 0).astype(jnp.bfloat16)
        ssB = ssB + (memB - _TH > 0).astype(jnp.bfloat16)
        return memA, ssA, memB, ssB

    _, ssA, _, ssB = jax.lax.fori_loop(
        1, _T, step, (memA, ssA, memB, ssB))
    o_ref[:_B_HALF] = ssA.astype(jnp.float32) * (1.0 / _T)
    o_ref[_B_HALF:] = ssB.astype(jnp.float32) * (1.0 / _T)


def kernel(x, W_in, W_rec):
    B = x.shape[0]
    x_p = jnp.pad(x, ((0, 0), (0, _K_PAD - _N_INPUT)))
    win_t = jnp.pad(W_in.T, ((0, _K_PAD - _N_INPUT), (0, _N_PAD - _N_RES)))
    wrec_t = jnp.pad(W_rec.T, ((0, _N_PAD - _N_RES), (0, _N_PAD - _N_RES)))

    out = pl.pallas_call(
        _lsm_body,
        out_shape=jax.ShapeDtypeStruct((B, _N_PAD), jnp.float32),
        grid=(B // _B_BLK,),
        in_specs=[
            pl.BlockSpec((_B_BLK, _K_PAD), lambda b: (b, 0)),
            pl.BlockSpec((_K_PAD, _N_PAD), lambda b: (0, 0)),
            pl.BlockSpec((_N_PAD, _N_PAD), lambda b: (0, 0)),
        ],
        out_specs=pl.BlockSpec((_B_BLK, _N_PAD), lambda b: (b, 0)),
        compiler_params=pltpu.CompilerParams(
            dimension_semantics=("parallel",),
        ),
        name="lsm_fused",
    )(x_p, win_t, wrec_t)
    return out[:, :_N_RES]


# speedup vs baseline: 1.8535x; 1.2211x over previous
import jax
import jax.numpy as jnp
from jax.experimental import pallas as pl
from jax.experimental.pallas import tpu as pltpu

_N_INPUT = 28 * 28
_N_RES = 1000
_T = 25
_BETA = 0.95
_TH = 1.0

_K_PAD = 896
_N_PAD = 1024
_B_BLK = 1024
_B_HALF = _B_BLK // 2


def _lsm_body(x_ref, win_ref, wrec_ref, o_ref):
    icA = jnp.dot(x_ref[:_B_HALF], win_ref[...],
                  preferred_element_type=jnp.float32)
    icB = jnp.dot(x_ref[_B_HALF:], win_ref[...],
                  preferred_element_type=jnp.float32)

    memA = icA
    ssA = (memA - _TH > 0).astype(jnp.bfloat16)
    memB = icB
    ssB = (memB - _TH > 0).astype(jnp.bfloat16)

    def one_step(memA, ssA, memB, ssB):
        spkA = (memA - _TH > 0).astype(jnp.float32)
        spkB = (memB - _TH > 0).astype(jnp.float32)
        recA = jnp.dot(spkA, wrec_ref[...], preferred_element_type=jnp.float32)
        recB = jnp.dot(spkB, wrec_ref[...], preferred_element_type=jnp.float32)
        memA = _BETA * memA + (icA + recA) - spkA * _TH
        memB = _BETA * memB + (icB + recB) - spkB * _TH
        ssA = ssA + (memA - _TH > 0).astype(jnp.bfloat16)
        ssB = ssB + (memB - _TH > 0).astype(jnp.bfloat16)
        return memA, ssA, memB, ssB

    def step2(_, c):
        return one_step(*one_step(*c))

    _, ssA, _, ssB = jax.lax.fori_loop(
        0, (_T - 1) // 2, step2, (memA, ssA, memB, ssB))
    o_ref[:_B_HALF] = ssA.astype(jnp.float32) * (1.0 / _T)
    o_ref[_B_HALF:] = ssB.astype(jnp.float32) * (1.0 / _T)


def kernel(x, W_in, W_rec):
    B = x.shape[0]
    x_p = jnp.pad(x, ((0, 0), (0, _K_PAD - _N_INPUT)))
    win_t = jnp.pad(W_in.T, ((0, _K_PAD - _N_INPUT), (0, _N_PAD - _N_RES)))
    wrec_t = jnp.pad(W_rec.T, ((0, _N_PAD - _N_RES), (0, _N_PAD - _N_RES)))

    out = pl.pallas_call(
        _lsm_body,
        out_shape=jax.ShapeDtypeStruct((B, _N_PAD), jnp.float32),
        grid=(B // _B_BLK,),
        in_specs=[
            pl.BlockSpec((_B_BLK, _K_PAD), lambda b: (b, 0)),
            pl.BlockSpec((_K_PAD, _N_PAD), lambda b: (0, 0)),
            pl.BlockSpec((_N_PAD, _N_PAD), lambda b: (0, 0)),
        ],
        out_specs=pl.BlockSpec((_B_BLK, _N_PAD), lambda b: (b, 0)),
        compiler_params=pltpu.CompilerParams(
            dimension_semantics=("parallel",),
        ),
        name="lsm_fused",
    )(x_p, win_t, wrec_t)
    return out[:, :_N_RES]


# speedup vs baseline: 1.9402x; 1.0468x over previous
import jax
import jax.numpy as jnp
from jax.experimental import pallas as pl
from jax.experimental.pallas import tpu as pltpu

_N_INPUT = 28 * 28
_N_RES = 1000
_T = 25
_BETA = 0.95
_TH = 1.0

_K_PAD = 896
_N_PAD = 1024
_B_BLK = 1024
_N_CHAINS = 4
_B_SUB = _B_BLK // _N_CHAINS


def _lsm_body(x_ref, win_ref, wrec_ref, o_ref, mem_ref, ss_ref, ic_ref):
    for i in range(_N_CHAINS):
        sl = slice(i * _B_SUB, (i + 1) * _B_SUB)
        ic = jnp.dot(x_ref[sl], win_ref[...],
                     preferred_element_type=jnp.float32)
        ic_ref[sl] = ic
        mem_ref[sl] = ic
        ss_ref[sl] = ((ic - _TH > 0)).astype(jnp.bfloat16)

    def one_step():
        for i in range(_N_CHAINS):
            sl = slice(i * _B_SUB, (i + 1) * _B_SUB)
            m = mem_ref[sl]
            spk = m - _TH > 0
            rec = jnp.dot(spk.astype(jnp.float32), wrec_ref[...],
                          preferred_element_type=jnp.float32)
            m = _BETA * m + (ic_ref[sl] + rec)
            mem_ref[sl] = m
            ss_ref[sl] = ss_ref[sl] + (m - _TH > 0).astype(jnp.bfloat16)

    def step2(_, c):
        one_step()
        one_step()
        return c

    jax.lax.fori_loop(0, (_T - 1) // 2, step2, 0)
    for i in range(_N_CHAINS):
        sl = slice(i * _B_SUB, (i + 1) * _B_SUB)
        o_ref[sl] = ss_ref[sl].astype(jnp.float32) * (1.0 / _T)


def kernel(x, W_in, W_rec):
    B = x.shape[0]
    x_p = jnp.pad(x, ((0, 0), (0, _K_PAD - _N_INPUT)))
    win_t = jnp.pad(W_in.T, ((0, _K_PAD - _N_INPUT), (0, _N_PAD - _N_RES)))
    wrec_t = jnp.pad(W_rec.T, ((0, _N_PAD - _N_RES), (0, _N_PAD - _N_RES)))
    wrec_t = wrec_t - _TH * jnp.eye(_N_PAD, dtype=jnp.float32)

    out = pl.pallas_call(
        _lsm_body,
        out_shape=jax.ShapeDtypeStruct((B, _N_PAD), jnp.float32),
        grid=(B // _B_BLK,),
        in_specs=[
            pl.BlockSpec((_B_BLK, _K_PAD), lambda b: (b, 0)),
            pl.BlockSpec((_K_PAD, _N_PAD), lambda b: (0, 0)),
            pl.BlockSpec((_N_PAD, _N_PAD), lambda b: (0, 0)),
        ],
        out_specs=pl.BlockSpec((_B_BLK, _N_PAD), lambda b: (b, 0)),
        scratch_shapes=[
            pltpu.VMEM((_B_BLK, _N_PAD), jnp.float32),
            pltpu.VMEM((_B_BLK, _N_PAD), jnp.bfloat16),
            pltpu.VMEM((_B_BLK, _N_PAD), jnp.float32),
        ],
        compiler_params=pltpu.CompilerParams(
            dimension_semantics=("parallel",),
        ),
        name="lsm_fused",
    )(x_p, win_t, wrec_t)
    return out[:, :_N_RES]
